# Initial kernel scaffold; baseline (speedup 1.0000x reference)
import jax
import jax.numpy as jnp
from jax.experimental import pallas as pl


def kernel(input, cell_1_mask, cell_2_mask, cell_1_bounds, cell_1_sizes, cell_2_sizes):
    raise NotImplementedError("write your pallas kernel here")


# speedup vs baseline: 14.8548x; 14.8548x over previous
import functools

import jax
import jax.numpy as jnp
from jax import lax
from jax.experimental import pallas as pl
from jax.experimental.pallas import tpu as pltpu
from jax.experimental.pallas import tpu_sc as plsc

N = 6400000
C1 = 100000
C2 = 50000
NT = 32
W1 = 3136
W2 = 1568
CH = 8192
L = 16
POS_INF = float("inf")
NEG_INF = float("-inf")

_mesh = lambda: plsc.VectorSubcoreMesh(core_axis_name="c", subcore_axis_name="s")


def _take(v, idx):
    return v.at[idx].get(mode="promise_in_bounds")


def _wid():
    return lax.axis_index("s") * 2 + lax.axis_index("c")


def _stats_body(x_hbm, lab1_hbm, lab2_hbm, sz1_hbm, sz2_hbm, st1_hbm, st2_hbm,
                out1_hbm, out2c_hbm,
                tab_s, tab_mn, tab_mx, rowbuf, wbuf, szbuf, stbuf, xbuf, labbuf):
    wid = _wid()
    i16 = lax.iota(jnp.int32, L)

    def run_job(lab_hbm, sz_hbm, st_hbm, W, interleave):
        lo = pl.multiple_of(wid * W, L)
        pltpu.sync_copy(st_hbm, stbuf)
        st_vec = stbuf[pl.ds(wid, L)]
        st = st_vec[0]
        en = st_vec[1]

        def init_body(i, _):
            o = i * L
            tab_s[pl.ds(o, L)] = jnp.zeros((L,), jnp.float32)
            tab_mn[pl.ds(o, L)] = jnp.full((L,), POS_INF, jnp.float32)
            tab_mx[pl.ds(o, L)] = jnp.full((L,), NEG_INF, jnp.float32)
            return 0

        lax.fori_loop(0, W // L, init_body, 0)

        s0 = st & ~(L - 1)
        nch = (en - s0 + (CH - 1)) // CH

        def chunk_body(k, _):
            logical = s0 + k * CH
            phys = pl.multiple_of(jnp.minimum(logical, N - CH), L)
            pltpu.sync_copy(x_hbm.at[pl.ds(phys, CH)], xbuf)
            pltpu.sync_copy(lab_hbm.at[pl.ds(phys, CH)], labbuf)
            glo = jnp.maximum(st, logical)

            def vec_body(i, _):
                o = i * L
                lab = labbuf[pl.ds(o, L)]
                x = xbuf[pl.ds(o, L)]
                gidx = phys + o + i16
                valid = (gidx >= glo) & (gidx < en)
                vs = jnp.where(valid, x, 0.0)
                vmn = jnp.where(valid, x, POS_INF)
                vmx = jnp.where(valid, x, NEG_INF)
                for d in (1, 2, 4, 8):
                    idxd = jnp.maximum(i16 - d, 0)
                    glab = _take(lab, idxd)
                    keep = (glab == lab) & (i16 >= d)
                    vs = vs + jnp.where(keep, _take(vs, idxd), 0.0)
                    vmn = jnp.minimum(vmn, jnp.where(keep, _take(vmn, idxd), POS_INF))
                    vmx = jnp.maximum(vmx, jnp.where(keep, _take(vmx, idxd), NEG_INF))
                nxt = _take(lab, jnp.minimum(i16 + 1, L - 1))
                is_last = (lab != nxt) | (i16 == L - 1)
                m = is_last & valid
                rel = jnp.clip(lab - lo, 0, W - 1)
                cs = plsc.load_gather(tab_s, [rel], mask=m)
                plsc.store_scatter(tab_s, [rel], cs + vs, mask=m)
                cn = plsc.load_gather(tab_mn, [rel], mask=m)
                plsc.store_scatter(tab_mn, [rel], jnp.minimum(cn, vmn), mask=m)
                cx = plsc.load_gather(tab_mx, [rel], mask=m)
                plsc.store_scatter(tab_mx, [rel], jnp.maximum(cx, vmx), mask=m)
                return 0

            lax.fori_loop(0, CH // L, vec_body, 0)
            return 0

        lax.fori_loop(0, nch, chunk_body, 0)

        pltpu.sync_copy(sz_hbm.at[pl.ds(lo, W)], szbuf.at[pl.ds(0, W)])

        def fin_body(r, _):
            o = r * L
            sv = tab_s[pl.ds(o, L)]
            mnv = tab_mn[pl.ds(o, L)]
            mxv = tab_mx[pl.ds(o, L)]
            szv = szbuf[pl.ds(o, L)]
            wv = jnp.exp(-szv.astype(jnp.float32)) - 0.5
            if interleave:
                cols = o * 4 + 4 * i16
                plsc.store_scatter(rowbuf, [cols], sv)
                plsc.store_scatter(rowbuf, [cols + 1], mnv)
                plsc.store_scatter(rowbuf, [cols + 2], mxv)
                plsc.store_scatter(rowbuf, [cols + 3], wv)
            else:
                wbuf[pl.ds(o, L)] = wv
            return 0

        lax.fori_loop(0, W // L, fin_body, 0)

        if interleave:
            o1 = pl.multiple_of(wid * (W1 * 4), L)
            pltpu.sync_copy(rowbuf, out1_hbm.at[pl.ds(o1, W1 * 4)])
        else:
            for ci, src_ref in enumerate((tab_s, tab_mn, tab_mx, wbuf)):
                oc = pl.multiple_of(ci * (NT * W2) + lo, L)
                pltpu.sync_copy(src_ref.at[pl.ds(0, W)], out2c_hbm.at[pl.ds(oc, W)])

    run_job(lab1_hbm, sz1_hbm, st1_hbm, W1, True)
    run_job(lab2_hbm, sz2_hbm, st2_hbm, W2, False)


_stats_call = functools.partial(
    pl.kernel,
    out_type=(
        jax.ShapeDtypeStruct((NT * W1 * 4,), jnp.float32),
        jax.ShapeDtypeStruct((4 * NT * W2,), jnp.float32),
    ),
    mesh=_mesh(),
    compiler_params=pltpu.CompilerParams(needs_layout_passes=False),
    scratch_types=[
        pltpu.VMEM((W1,), jnp.float32),
        pltpu.VMEM((W1,), jnp.float32),
        pltpu.VMEM((W1,), jnp.float32),
        pltpu.VMEM((W1 * 4,), jnp.float32),
        pltpu.VMEM((W2,), jnp.float32),
        pltpu.VMEM((W1,), jnp.int32),
        pltpu.VMEM((48,), jnp.int32),
        pltpu.VMEM((CH,), jnp.float32),
        pltpu.VMEM((CH,), jnp.int32),
    ],
)(_stats_body)


def _gather_body(cols_hbm, bu_hbm, bv_hbm, outu_hbm, outv_hbm,
                 colbuf, bbuf, uidx, vidx, urows, vrows):
    wid = _wid()
    i16 = lax.iota(jnp.int32, L)
    base = pl.multiple_of(wid * W1, L)

    def fill(idxbuf):
        def body(j, _):
            o = j * L
            b = bbuf[pl.ds(o, L)]
            u = b - 1
            u = jnp.where(u < 0, u + C2, u)
            u = jnp.clip(u, 0, C2 - 1)
            idxbuf[pl.ds(o, L)] = u
            return 0

        lax.fori_loop(0, W1 // L, body, 0)

    pltpu.sync_copy(bu_hbm.at[pl.ds(base, W1)], bbuf)
    fill(uidx)
    pltpu.sync_copy(bv_hbm.at[pl.ds(base, W1)], bbuf)
    fill(vidx)

    for c in range(4):
        pltpu.sync_copy(cols_hbm.at[pl.ds(c * (NT * W2), NT * W2)], colbuf)

        def gbody(j, _):
            o = j * L
            tgt = o * 4 + 4 * i16 + c
            gu = plsc.load_gather(colbuf, [uidx[pl.ds(o, L)]])
            plsc.store_scatter(urows, [tgt], gu)
            gv = plsc.load_gather(colbuf, [vidx[pl.ds(o, L)]])
            plsc.store_scatter(vrows, [tgt], gv)
            return 0

        lax.fori_loop(0, W1 // L, gbody, 0)

    ob = pl.multiple_of(wid * (W1 * 4), L)
    pltpu.sync_copy(urows, outu_hbm.at[pl.ds(ob, W1 * 4)])
    pltpu.sync_copy(vrows, outv_hbm.at[pl.ds(ob, W1 * 4)])


_gather_call = functools.partial(
    pl.kernel,
    out_type=(
        jax.ShapeDtypeStruct((NT * W1 * 4,), jnp.float32),
        jax.ShapeDtypeStruct((NT * W1 * 4,), jnp.float32),
    ),
    mesh=_mesh(),
    compiler_params=pltpu.CompilerParams(needs_layout_passes=False),
    scratch_types=[
        pltpu.VMEM((NT * W2,), jnp.float32),
        pltpu.VMEM((W1,), jnp.int32),
        pltpu.VMEM((W1,), jnp.int32),
        pltpu.VMEM((W1,), jnp.int32),
        pltpu.VMEM((W1 * 4,), jnp.float32),
        pltpu.VMEM((W1 * 4,), jnp.float32),
    ],
)(_gather_body)


def kernel(input, cell_1_mask, cell_2_mask, cell_1_bounds, cell_1_sizes, cell_2_sizes):
    q1 = jnp.arange(NT + 1, dtype=jnp.int32) * W1
    q2 = jnp.arange(NT + 1, dtype=jnp.int32) * W2
    st1 = jnp.pad(jnp.searchsorted(cell_1_mask, q1, side="left").astype(jnp.int32), (0, 15))
    st2 = jnp.pad(jnp.searchsorted(cell_2_mask, q2, side="left").astype(jnp.int32), (0, 15))
    sz1 = jnp.pad(cell_1_sizes, (0, NT * W1 - C1))
    sz2 = jnp.pad(cell_2_sizes, (0, NT * W2 - C2))
    out1, out2c = _stats_call(input, cell_1_mask, cell_2_mask, sz1, sz2, st1, st2)
    cell_1_stats = out1.reshape(NT * W1, 4)[:C1]
    bu = jnp.pad(cell_1_bounds[:, 0], (0, NT * W1 - C1))
    bv = jnp.pad(cell_1_bounds[:, 1], (0, NT * W1 - C1))
    su, sv = _gather_call(out2c, bu, bv)
    stats_u = su.reshape(NT * W1, 4)[:C1]
    stats_v = sv.reshape(NT * W1, 4)[:C1]
    return (cell_1_stats, stats_u, stats_v)
